# baseline (speedup 1.0000x reference)
import functools

import jax
import jax.numpy as jnp
from jax import lax
from jax.experimental import pallas as pl
from jax.experimental.pallas import tpu as pltpu
from jax.experimental.pallas import tpu_sc as plsc

B = 16384
F = 26
K = 32
NC = 2
NS = 16
NW = NC * NS
BPW = B // NW
C = 32
NCHUNK = BPW // C
L = 16
TW = 128

_mesh = plsc.VectorSubcoreMesh(core_axis_name="c", subcore_axis_name="s")


@functools.partial(
    pl.kernel,
    out_type=jax.ShapeDtypeStruct((B * K,), jnp.float32),
    mesh=_mesh,
    compiler_params=pltpu.CompilerParams(use_tc_tiling_on_sc=True),
    scratch_types=[
        pltpu.VMEM((C * F,), jnp.int32),
        pltpu.VMEM((C * K,), jnp.int32),
        pltpu.VMEM((C * K,), jnp.float32),
        pltpu.VMEM((C * F, TW), jnp.float32),
        pltpu.VMEM((C * K,), jnp.float32),
        pltpu.SemaphoreType.DMA,
    ],
)
def _fm_second_order(vals_hbm, idx_hbm, sub_hbm, table_hbm, out_hbm,
                     idx_v, sub_v, vals_v, rows_v, out_v, sem):
    wid = lax.axis_index("s") * NC + lax.axis_index("c")
    base = wid * BPW

    def chunk_body(g, carry):
        row0 = base + g * C
        pltpu.sync_copy(idx_hbm.at[pl.ds(row0 * F, C * F)], idx_v)
        pltpu.sync_copy(sub_hbm.at[pl.ds(row0 * K, C * K)], sub_v)
        pltpu.sync_copy(vals_hbm.at[pl.ds(row0 * K, C * K)], vals_v)
        pltpu.async_copy(table_hbm.at[idx_v], rows_v, sem).wait()

        def b_body(b, carry2):
            a1_lo = jnp.zeros((L,), jnp.float32)
            a1_hi = jnp.zeros((L,), jnp.float32)
            a2_lo = jnp.zeros((L,), jnp.float32)
            a2_hi = jnp.zeros((L,), jnp.float32)
            j0 = b * F
            v_lo = vals_v[pl.ds(b * K, L)]
            v_hi = vals_v[pl.ds(b * K + L, L)]
            s_lo = sub_v[pl.ds(b * K, L)]
            s_hi = sub_v[pl.ds(b * K + L, L)]
            for f in range(F):
                w = v_lo[f] if f < L else v_hi[f - L]
                sub = s_lo[f] if f < L else s_hi[f - L]
                e_lo = rows_v[j0 + f, pl.ds(sub, L)]
                e_hi = rows_v[j0 + f, pl.ds(sub + L, L)]
                we_lo = w * e_lo
                we_hi = w * e_hi
                a1_lo = a1_lo + we_lo
                a1_hi = a1_hi + we_hi
                a2_lo = a2_lo + we_lo * we_lo
                a2_hi = a2_hi + we_hi * we_hi
            out_v[pl.ds(b * K, L)] = a1_lo * a1_lo - a2_lo
            out_v[pl.ds(b * K + L, L)] = a1_hi * a1_hi - a2_hi
            return carry2

        lax.fori_loop(0, C, b_body, 0)
        pltpu.sync_copy(out_v, out_hbm.at[pl.ds(row0 * K, C * K)])
        return carry

    lax.fori_loop(0, NCHUNK, chunk_body, 0)


def kernel(feature_values, feature_idx, feature_embeddings):
    vals_flat = jnp.pad(feature_values, ((0, 0), (0, K - F))).reshape(-1)
    tile_idx = (feature_idx >> 2).reshape(-1)
    sub_off = jnp.pad((feature_idx & 3) * K, ((0, 0), (0, K - F))).reshape(-1)
    table_tiles = jnp.concatenate(
        [feature_embeddings[i::4] for i in range(4)], axis=1)
    out_flat = _fm_second_order(vals_flat, tile_idx, sub_off, table_tiles)
    return out_flat.reshape(B, K)


# speedup vs baseline: 5.5851x; 5.5851x over previous
import functools

import jax
import jax.numpy as jnp
from jax import lax
from jax.experimental import pallas as pl
from jax.experimental.pallas import tpu as pltpu
from jax.experimental.pallas import tpu_sc as plsc

B = 16384
F = 26
K = 32
V = 1000000
NC = 2
NS = 16
NW = NC * NS
BPW = B // NW
C = 64
NCHUNK = BPW // C
L = 16

CW = 512
NFULL = V // CW
CPW = NFULL // NW
TAIL = V - NFULL * CW

_mesh = plsc.VectorSubcoreMesh(core_axis_name="c", subcore_axis_name="s")


@functools.partial(
    pl.kernel,
    out_type=jax.ShapeDtypeStruct((V * K,), jnp.float32),
    mesh=_mesh,
    compiler_params=pltpu.CompilerParams(
        use_tc_tiling_on_sc=True, needs_layout_passes=False),
    scratch_types=[
        pltpu.VMEM((K, CW), jnp.float32),
        pltpu.VMEM((CW * K,), jnp.float32),
        pltpu.VMEM((TAIL * K,), jnp.float32),
    ],
)
def _table_to_rowmajor(tt_hbm, tail_hbm, out_hbm, in_v, out_v, tail_v):
    wid = lax.axis_index("s") * NC + lax.axis_index("c")
    lanes = lax.iota(jnp.int32, L) * K

    def do_block(c0):
        pltpu.sync_copy(tt_hbm.at[:, pl.ds(c0, CW)], in_v)

        def m_body(m, carry):
            base = m * (L * K)
            for k in range(K):
                vec = in_v[k, pl.ds(m * L, L)]
                plsc.store_scatter(out_v, [lanes + (base + k)], vec)
            return carry

        lax.fori_loop(0, CW // L, m_body, 0)
        pltpu.sync_copy(out_v, out_hbm.at[pl.ds(c0 * K, CW * K)])

    def chunk_body(i, carry):
        do_block((wid * CPW + i) * CW)
        return carry

    lax.fori_loop(0, CPW, chunk_body, 0)

    @pl.when(wid == 0)
    def _extra():
        do_block((NFULL - 1) * CW)

    @pl.when(wid == 1)
    def _tail():
        pltpu.sync_copy(tail_hbm, tail_v)
        pltpu.sync_copy(tail_v, out_hbm.at[pl.ds(NFULL * CW * K, TAIL * K)])


@functools.partial(
    pl.kernel,
    out_type=jax.ShapeDtypeStruct((B, K), jnp.float32),
    mesh=_mesh,
    compiler_params=pltpu.CompilerParams(use_tc_tiling_on_sc=False),
    scratch_types=[
        pltpu.VMEM((C * F,), jnp.int32),
        pltpu.VMEM((C, K), jnp.float32),
        pltpu.VMEM((C * F, K), jnp.float32),
        pltpu.VMEM((C, K), jnp.float32),
        pltpu.SemaphoreType.DMA,
    ],
)
def _fm_second_order(vals_hbm, idx_hbm, table_hbm, out_hbm,
                     idx_v, vals_v, rows_v, out_v, sem):
    wid = lax.axis_index("s") * NC + lax.axis_index("c")
    base = wid * BPW

    def chunk_body(g, carry):
        row0 = base + g * C
        flat0 = row0 * F
        pltpu.sync_copy(idx_hbm.at[pl.ds(flat0, C * F)], idx_v)
        pltpu.sync_copy(vals_hbm.at[pl.ds(row0, C)], vals_v)
        pltpu.async_copy(table_hbm.at[idx_v], rows_v, sem).wait()

        def b_body(b, carry2):
            a1_lo = jnp.zeros((L,), jnp.float32)
            a1_hi = jnp.zeros((L,), jnp.float32)
            a2_lo = jnp.zeros((L,), jnp.float32)
            a2_hi = jnp.zeros((L,), jnp.float32)
            j0 = b * F
            v_lo = vals_v[b, pl.ds(0, L)]
            v_hi = vals_v[b, pl.ds(L, L)]
            for f in range(F):
                w = v_lo[f] if f < L else v_hi[f - L]
                e_lo = rows_v[j0 + f, pl.ds(0, L)]
                e_hi = rows_v[j0 + f, pl.ds(L, L)]
                we_lo = w * e_lo
                we_hi = w * e_hi
                a1_lo = a1_lo + we_lo
                a1_hi = a1_hi + we_hi
                a2_lo = a2_lo + we_lo * we_lo
                a2_hi = a2_hi + we_hi * we_hi
            out_v[b, pl.ds(0, L)] = a1_lo * a1_lo - a2_lo
            out_v[b, pl.ds(L, L)] = a1_hi * a1_hi - a2_hi
            return carry2

        lax.fori_loop(0, C, b_body, 0)
        pltpu.sync_copy(out_v, out_hbm.at[pl.ds(row0, C)])
        return carry

    lax.fori_loop(0, NCHUNK, chunk_body, 0)


def kernel(feature_values, feature_idx, feature_embeddings):
    vals_padded = jnp.pad(feature_values, ((0, 0), (0, K - F)))
    table_flat = _table_to_rowmajor(
        feature_embeddings.T,
        feature_embeddings[NFULL * CW:].reshape(-1),
    )
    return _fm_second_order(
        vals_padded,
        feature_idx.reshape(-1),
        table_flat.reshape(V, K),
    )


# speedup vs baseline: 6.5829x; 1.1786x over previous
import functools

import jax
import jax.numpy as jnp
from jax import lax
from jax.experimental import pallas as pl
from jax.experimental.pallas import tpu as pltpu
from jax.experimental.pallas import tpu_sc as plsc

B = 16384
F = 26
K = 32
V = 1000000
NC = 2
NS = 16
NW = NC * NS
BPW = B // NW
C = 64
NCHUNK = BPW // C
L = 16

CW = 512
NFULL = V // CW
CPW = NFULL // NW
TAIL = V - NFULL * CW

_mesh = plsc.VectorSubcoreMesh(core_axis_name="c", subcore_axis_name="s")


@functools.partial(
    pl.kernel,
    out_type=jax.ShapeDtypeStruct((V * K,), jnp.float32),
    mesh=_mesh,
    compiler_params=pltpu.CompilerParams(
        use_tc_tiling_on_sc=True, needs_layout_passes=False),
    scratch_types=[
        pltpu.VMEM((K, CW), jnp.float32),
        pltpu.VMEM((K, CW), jnp.float32),
        pltpu.VMEM((CW * K,), jnp.float32),
        pltpu.VMEM((CW * K,), jnp.float32),
        pltpu.VMEM((TAIL * K,), jnp.float32),
        pltpu.SemaphoreType.DMA,
        pltpu.SemaphoreType.DMA,
        pltpu.SemaphoreType.DMA,
        pltpu.SemaphoreType.DMA,
    ],
)
def _table_to_rowmajor(tt_hbm, tail_hbm, out_hbm, in_v0, in_v1,
                       out_v0, out_v1, tail_v,
                       in_sem0, in_sem1, out_sem0, out_sem1):
    wid = lax.axis_index("s") * NC + lax.axis_index("c")
    lanes32 = lax.iota(jnp.int32, L) * K
    in_bufs = (in_v0, in_v1)
    out_bufs = (out_v0, out_v1)
    in_sems = (in_sem0, in_sem1)
    out_sems = (out_sem0, out_sem1)

    def col0(i):
        return (wid * CPW + i) * CW

    def in_copy(s, i):
        return pltpu.make_async_copy(
            tt_hbm.at[:, pl.ds(col0(i), CW)], in_bufs[s], in_sems[s])

    def out_copy(s, i):
        return pltpu.make_async_copy(
            out_bufs[s], out_hbm.at[pl.ds(col0(i) * K, CW * K)], out_sems[s])

    def compute(s):
        def m_body(m, carry):
            idxm = lanes32 + m * (L * K)
            for k in range(K):
                vec = in_bufs[s][k, pl.ds(m * L, L)]
                plsc.store_scatter(out_bufs[s], [idxm + k], vec)
            return carry

        lax.fori_loop(0, CW // L, m_body, 0)

    in_copy(0, 0).start()
    in_copy(1, 1).start()

    def pair_body(j, carry):
        for s in range(2):
            i = 2 * j + s
            in_copy(s, i).wait()

            @pl.when(j > 0)
            def _drain():
                out_copy(s, i).wait()

            compute(s)
            out_copy(s, i).start()

            @pl.when(i + 2 <= CPW - 1)
            def _prefetch():
                in_copy(s, i + 2).start()
        return carry

    lax.fori_loop(0, (CPW - 1) // 2, pair_body, 0)

    last = CPW - 1
    in_copy(0, last).wait()
    out_copy(0, last).wait()
    compute(0)
    out_copy(0, last).start()

    @pl.when(wid == 0)
    def _extra():
        out_copy(1, 0).wait()
        ex = NW * CPW
        pltpu.sync_copy(tt_hbm.at[:, pl.ds(ex * CW, CW)], in_v1)
        compute(1)
        pltpu.async_copy(out_v1,
                         out_hbm.at[pl.ds(ex * CW * K, CW * K)],
                         out_sems[1]).wait()

    @pl.when(wid == 1)
    def _tail():
        pltpu.sync_copy(tail_hbm, tail_v)
        pltpu.sync_copy(tail_v, out_hbm.at[pl.ds(NFULL * CW * K, TAIL * K)])

    out_copy(0, last).wait()

    @pl.when(wid != 0)
    def _drain1():
        out_copy(1, 0).wait()


@functools.partial(
    pl.kernel,
    out_type=jax.ShapeDtypeStruct((B, K), jnp.float32),
    mesh=_mesh,
    compiler_params=pltpu.CompilerParams(use_tc_tiling_on_sc=False),
    scratch_types=[
        pltpu.VMEM((C * F,), jnp.int32),
        pltpu.VMEM((C, K), jnp.float32),
        pltpu.VMEM((C * F, K), jnp.float32),
        pltpu.VMEM((C, K), jnp.float32),
        pltpu.SemaphoreType.DMA,
    ],
)
def _fm_second_order(vals_hbm, idx_hbm, table_hbm, out_hbm,
                     idx_v, vals_v, rows_v, out_v, sem):
    wid = lax.axis_index("s") * NC + lax.axis_index("c")
    base = wid * BPW

    def chunk_body(g, carry):
        row0 = base + g * C
        flat0 = row0 * F
        pltpu.sync_copy(idx_hbm.at[pl.ds(flat0, C * F)], idx_v)
        pltpu.sync_copy(vals_hbm.at[pl.ds(row0, C)], vals_v)
        pltpu.async_copy(table_hbm.at[idx_v], rows_v, sem).wait()

        def b_body(b, carry2):
            a1_lo = jnp.zeros((L,), jnp.float32)
            a1_hi = jnp.zeros((L,), jnp.float32)
            a2_lo = jnp.zeros((L,), jnp.float32)
            a2_hi = jnp.zeros((L,), jnp.float32)
            j0 = b * F
            v_lo = vals_v[b, pl.ds(0, L)]
            v_hi = vals_v[b, pl.ds(L, L)]
            for f in range(F):
                w = v_lo[f] if f < L else v_hi[f - L]
                e_lo = rows_v[j0 + f, pl.ds(0, L)]
                e_hi = rows_v[j0 + f, pl.ds(L, L)]
                we_lo = w * e_lo
                we_hi = w * e_hi
                a1_lo = a1_lo + we_lo
                a1_hi = a1_hi + we_hi
                a2_lo = a2_lo + we_lo * we_lo
                a2_hi = a2_hi + we_hi * we_hi
            out_v[b, pl.ds(0, L)] = a1_lo * a1_lo - a2_lo
            out_v[b, pl.ds(L, L)] = a1_hi * a1_hi - a2_hi
            return carry2

        lax.fori_loop(0, C, b_body, 0)
        pltpu.sync_copy(out_v, out_hbm.at[pl.ds(row0, C)])
        return carry

    lax.fori_loop(0, NCHUNK, chunk_body, 0)


def kernel(feature_values, feature_idx, feature_embeddings):
    vals_padded = jnp.pad(feature_values, ((0, 0), (0, K - F)))
    table_flat = _table_to_rowmajor(
        feature_embeddings.T,
        feature_embeddings[NFULL * CW:].reshape(-1),
    )
    return _fm_second_order(
        vals_padded,
        feature_idx.reshape(-1),
        table_flat.reshape(V, K),
    )


# speedup vs baseline: 8.2263x; 1.2497x over previous
import functools

import jax
import jax.numpy as jnp
from jax import lax
from jax.experimental import pallas as pl
from jax.experimental.pallas import tpu as pltpu
from jax.experimental.pallas import tpu_sc as plsc

B = 16384
F = 26
K = 32
V = 1000000
NC = 2
NS = 16
NW = NC * NS
BPW = B // NW
C = 64
NCHUNK = BPW // C
L = 16

CW = 512
NFULL = V // CW
CPW = NFULL // NW
TAIL = V - NFULL * CW

_mesh = plsc.VectorSubcoreMesh(core_axis_name="c", subcore_axis_name="s")


@functools.partial(
    pl.kernel,
    out_type=jax.ShapeDtypeStruct((V * K,), jnp.float32),
    mesh=_mesh,
    compiler_params=pltpu.CompilerParams(
        use_tc_tiling_on_sc=True, needs_layout_passes=False),
    scratch_types=[
        pltpu.VMEM((K, CW), jnp.float32),
        pltpu.VMEM((K, CW), jnp.float32),
        pltpu.VMEM((CW * K,), jnp.float32),
        pltpu.VMEM((CW * K,), jnp.float32),
        pltpu.VMEM((TAIL * K,), jnp.float32),
        pltpu.SemaphoreType.DMA,
        pltpu.SemaphoreType.DMA,
        pltpu.SemaphoreType.DMA,
        pltpu.SemaphoreType.DMA,
    ],
)
def _table_to_rowmajor(tt_hbm, tail_hbm, out_hbm, in_v0, in_v1,
                       out_v0, out_v1, tail_v,
                       in_sem0, in_sem1, out_sem0, out_sem1):
    wid = lax.axis_index("s") * NC + lax.axis_index("c")
    lanes32 = lax.iota(jnp.int32, L) * K
    in_bufs = (in_v0, in_v1)
    out_bufs = (out_v0, out_v1)
    in_sems = (in_sem0, in_sem1)
    out_sems = (out_sem0, out_sem1)

    def col0(i):
        return (wid * CPW + i) * CW

    def in_copy(s, i):
        return pltpu.make_async_copy(
            tt_hbm.at[:, pl.ds(col0(i), CW)], in_bufs[s], in_sems[s])

    def out_copy(s, i):
        return pltpu.make_async_copy(
            out_bufs[s], out_hbm.at[pl.ds(col0(i) * K, CW * K)], out_sems[s])

    def compute(s):
        def m_body(m, carry):
            idxm = lanes32 + m * (L * K)
            for k0 in range(0, K, 8):
                vecs = [in_bufs[s][k0 + d, pl.ds(m * L, L)] for d in range(8)]
                for d in range(8):
                    plsc.store_scatter(out_bufs[s], [idxm + (k0 + d)], vecs[d])
            return carry

        lax.fori_loop(0, CW // L, m_body, 0)

    in_copy(0, 0).start()
    in_copy(1, 1).start()

    def pair_body(j, carry):
        for s in range(2):
            i = 2 * j + s
            in_copy(s, i).wait()

            @pl.when(j > 0)
            def _drain():
                out_copy(s, i).wait()

            compute(s)
            out_copy(s, i).start()

            @pl.when(i + 2 <= CPW - 1)
            def _prefetch():
                in_copy(s, i + 2).start()
        return carry

    lax.fori_loop(0, (CPW - 1) // 2, pair_body, 0)

    last = CPW - 1
    in_copy(0, last).wait()
    out_copy(0, last).wait()
    compute(0)
    out_copy(0, last).start()

    @pl.when(wid == 0)
    def _extra():
        out_copy(1, 0).wait()
        ex = NW * CPW
        pltpu.sync_copy(tt_hbm.at[:, pl.ds(ex * CW, CW)], in_v1)
        compute(1)
        pltpu.async_copy(out_v1,
                         out_hbm.at[pl.ds(ex * CW * K, CW * K)],
                         out_sems[1]).wait()

    @pl.when(wid == 1)
    def _tail():
        pltpu.sync_copy(tail_hbm, tail_v)
        pltpu.sync_copy(tail_v, out_hbm.at[pl.ds(NFULL * CW * K, TAIL * K)])

    out_copy(0, last).wait()

    @pl.when(wid != 0)
    def _drain1():
        out_copy(1, 0).wait()


@functools.partial(
    pl.kernel,
    out_type=jax.ShapeDtypeStruct((B, K), jnp.float32),
    mesh=_mesh,
    compiler_params=pltpu.CompilerParams(use_tc_tiling_on_sc=False),
    scratch_types=[
        pltpu.VMEM((C * F,), jnp.int32),
        pltpu.VMEM((C, K), jnp.float32),
        pltpu.VMEM((C * F, K), jnp.float32),
        pltpu.VMEM((C, K), jnp.float32),
        pltpu.SemaphoreType.DMA,
    ],
)
def _fm_second_order(vals_hbm, idx_hbm, table_hbm, out_hbm,
                     idx_v, vals_v, rows_v, out_v, sem):
    wid = lax.axis_index("s") * NC + lax.axis_index("c")
    base = wid * BPW

    def chunk_body(g, carry):
        row0 = base + g * C
        flat0 = row0 * F
        pltpu.sync_copy(idx_hbm.at[pl.ds(flat0, C * F)], idx_v)
        pltpu.sync_copy(vals_hbm.at[pl.ds(row0, C)], vals_v)
        pltpu.async_copy(table_hbm.at[idx_v], rows_v, sem).wait()

        def b_body(b, carry2):
            a1_lo = jnp.zeros((L,), jnp.float32)
            a1_hi = jnp.zeros((L,), jnp.float32)
            a2_lo = jnp.zeros((L,), jnp.float32)
            a2_hi = jnp.zeros((L,), jnp.float32)
            j0 = b * F
            v_lo = vals_v[b, pl.ds(0, L)]
            v_hi = vals_v[b, pl.ds(L, L)]
            for f in range(F):
                w = v_lo[f] if f < L else v_hi[f - L]
                e_lo = rows_v[j0 + f, pl.ds(0, L)]
                e_hi = rows_v[j0 + f, pl.ds(L, L)]
                we_lo = w * e_lo
                we_hi = w * e_hi
                a1_lo = a1_lo + we_lo
                a1_hi = a1_hi + we_hi
                a2_lo = a2_lo + we_lo * we_lo
                a2_hi = a2_hi + we_hi * we_hi
            out_v[b, pl.ds(0, L)] = a1_lo * a1_lo - a2_lo
            out_v[b, pl.ds(L, L)] = a1_hi * a1_hi - a2_hi
            return carry2

        lax.fori_loop(0, C, b_body, 0)
        pltpu.sync_copy(out_v, out_hbm.at[pl.ds(row0, C)])
        return carry

    lax.fori_loop(0, NCHUNK, chunk_body, 0)


def kernel(feature_values, feature_idx, feature_embeddings):
    vals_padded = jnp.pad(feature_values, ((0, 0), (0, K - F)))
    table_flat = _table_to_rowmajor(
        feature_embeddings.T,
        feature_embeddings[NFULL * CW:].reshape(-1),
    )
    return _fm_second_order(
        vals_padded,
        feature_idx.reshape(-1),
        table_flat.reshape(V, K),
    )


# speedup vs baseline: 8.2266x; 1.0000x over previous
import functools

import jax
import jax.numpy as jnp
from jax import lax
from jax.experimental import pallas as pl
from jax.experimental.pallas import tpu as pltpu
from jax.experimental.pallas import tpu_sc as plsc

B = 16384
F = 26
K = 32
V = 1000000
NC = 2
NS = 16
NW = NC * NS
BPW = B // NW
C = 64
NCHUNK = BPW // C
L = 16

CW = 512
NFULL = V // CW
CPW = NFULL // NW
TAIL = V - NFULL * CW

_mesh = plsc.VectorSubcoreMesh(core_axis_name="c", subcore_axis_name="s")


@functools.partial(
    pl.kernel,
    out_type=jax.ShapeDtypeStruct((V * K,), jnp.float32),
    mesh=_mesh,
    compiler_params=pltpu.CompilerParams(
        use_tc_tiling_on_sc=True, needs_layout_passes=False),
    scratch_types=[
        pltpu.VMEM((K, CW), jnp.float32),
        pltpu.VMEM((K, CW), jnp.float32),
        pltpu.VMEM((CW * K,), jnp.float32),
        pltpu.VMEM((CW * K,), jnp.float32),
        pltpu.VMEM((TAIL * K,), jnp.float32),
        pltpu.SemaphoreType.DMA,
        pltpu.SemaphoreType.DMA,
        pltpu.SemaphoreType.DMA,
        pltpu.SemaphoreType.DMA,
    ],
)
def _table_to_rowmajor(tt_hbm, tail_hbm, out_hbm, in_v0, in_v1,
                       out_v0, out_v1, tail_v,
                       in_sem0, in_sem1, out_sem0, out_sem1):
    wid = lax.axis_index("s") * NC + lax.axis_index("c")
    lanes32 = lax.iota(jnp.int32, L) * K
    in_bufs = (in_v0, in_v1)
    out_bufs = (out_v0, out_v1)
    in_sems = (in_sem0, in_sem1)
    out_sems = (out_sem0, out_sem1)

    def col0(i):
        return (wid * CPW + i) * CW

    def in_copy(s, i):
        return pltpu.make_async_copy(
            tt_hbm.at[:, pl.ds(col0(i), CW)], in_bufs[s], in_sems[s])

    def out_copy(s, i):
        return pltpu.make_async_copy(
            out_bufs[s], out_hbm.at[pl.ds(col0(i) * K, CW * K)], out_sems[s])

    def compute(s):
        def m_body(m, carry):
            idxm = lanes32 + m * (L * K)
            for k0 in range(0, K, 16):
                vecs = [in_bufs[s][k0 + d, pl.ds(m * L, L)] for d in range(16)]
                for d in range(16):
                    plsc.store_scatter(out_bufs[s], [idxm + (k0 + d)], vecs[d])
            return carry

        lax.fori_loop(0, CW // L, m_body, 0)

    in_copy(0, 0).start()
    in_copy(1, 1).start()

    def pair_body(j, carry):
        for s in range(2):
            i = 2 * j + s
            in_copy(s, i).wait()

            @pl.when(j > 0)
            def _drain():
                out_copy(s, i).wait()

            compute(s)
            out_copy(s, i).start()

            @pl.when(i + 2 <= CPW - 1)
            def _prefetch():
                in_copy(s, i + 2).start()
        return carry

    lax.fori_loop(0, (CPW - 1) // 2, pair_body, 0)

    last = CPW - 1
    in_copy(0, last).wait()
    out_copy(0, last).wait()
    compute(0)
    out_copy(0, last).start()

    @pl.when(wid == 0)
    def _extra():
        out_copy(1, 0).wait()
        ex = NW * CPW
        pltpu.sync_copy(tt_hbm.at[:, pl.ds(ex * CW, CW)], in_v1)
        compute(1)
        pltpu.async_copy(out_v1,
                         out_hbm.at[pl.ds(ex * CW * K, CW * K)],
                         out_sems[1]).wait()

    @pl.when(wid == 1)
    def _tail():
        pltpu.sync_copy(tail_hbm, tail_v)
        pltpu.sync_copy(tail_v, out_hbm.at[pl.ds(NFULL * CW * K, TAIL * K)])

    out_copy(0, last).wait()

    @pl.when(wid != 0)
    def _drain1():
        out_copy(1, 0).wait()


@functools.partial(
    pl.kernel,
    out_type=jax.ShapeDtypeStruct((B, K), jnp.float32),
    mesh=_mesh,
    compiler_params=pltpu.CompilerParams(use_tc_tiling_on_sc=False),
    scratch_types=[
        pltpu.VMEM((C * F,), jnp.int32),
        pltpu.VMEM((C, K), jnp.float32),
        pltpu.VMEM((C * F, K), jnp.float32),
        pltpu.VMEM((C, K), jnp.float32),
        pltpu.SemaphoreType.DMA,
    ],
)
def _fm_second_order(vals_hbm, idx_hbm, table_hbm, out_hbm,
                     idx_v, vals_v, rows_v, out_v, sem):
    wid = lax.axis_index("s") * NC + lax.axis_index("c")
    base = wid * BPW

    def chunk_body(g, carry):
        row0 = base + g * C
        flat0 = row0 * F
        pltpu.sync_copy(idx_hbm.at[pl.ds(flat0, C * F)], idx_v)
        pltpu.sync_copy(vals_hbm.at[pl.ds(row0, C)], vals_v)
        pltpu.async_copy(table_hbm.at[idx_v], rows_v, sem).wait()

        def b_body(b, carry2):
            a1_lo = jnp.zeros((L,), jnp.float32)
            a1_hi = jnp.zeros((L,), jnp.float32)
            a2_lo = jnp.zeros((L,), jnp.float32)
            a2_hi = jnp.zeros((L,), jnp.float32)
            j0 = b * F
            v_lo = vals_v[b, pl.ds(0, L)]
            v_hi = vals_v[b, pl.ds(L, L)]
            for f in range(F):
                w = v_lo[f] if f < L else v_hi[f - L]
                e_lo = rows_v[j0 + f, pl.ds(0, L)]
                e_hi = rows_v[j0 + f, pl.ds(L, L)]
                we_lo = w * e_lo
                we_hi = w * e_hi
                a1_lo = a1_lo + we_lo
                a1_hi = a1_hi + we_hi
                a2_lo = a2_lo + we_lo * we_lo
                a2_hi = a2_hi + we_hi * we_hi
            out_v[b, pl.ds(0, L)] = a1_lo * a1_lo - a2_lo
            out_v[b, pl.ds(L, L)] = a1_hi * a1_hi - a2_hi
            return carry2

        lax.fori_loop(0, C, b_body, 0)
        pltpu.sync_copy(out_v, out_hbm.at[pl.ds(row0, C)])
        return carry

    lax.fori_loop(0, NCHUNK, chunk_body, 0)


def kernel(feature_values, feature_idx, feature_embeddings):
    vals_padded = jnp.pad(feature_values, ((0, 0), (0, K - F)))
    table_flat = _table_to_rowmajor(
        feature_embeddings.T,
        feature_embeddings[NFULL * CW:].reshape(-1),
    )
    return _fm_second_order(
        vals_padded,
        feature_idx.reshape(-1),
        table_flat.reshape(V, K),
    )


# speedup vs baseline: 8.5157x; 1.0351x over previous
import functools

import jax
import jax.numpy as jnp
from jax import lax
from jax.experimental import pallas as pl
from jax.experimental.pallas import tpu as pltpu
from jax.experimental.pallas import tpu_sc as plsc

B = 16384
F = 26
K = 32
V = 1000000
NC = 2
NS = 16
NW = NC * NS
BPW = B // NW
C = 64
NCHUNK = BPW // C
L = 16

CW = 512
NFULL = V // CW
CPW = NFULL // NW
TAIL = V - NFULL * CW

_mesh = plsc.VectorSubcoreMesh(core_axis_name="c", subcore_axis_name="s")


@functools.partial(
    pl.kernel,
    out_type=jax.ShapeDtypeStruct((V * K,), jnp.float32),
    mesh=_mesh,
    compiler_params=pltpu.CompilerParams(
        use_tc_tiling_on_sc=True, needs_layout_passes=False),
    scratch_types=[
        pltpu.VMEM((K, CW), jnp.float32),
        pltpu.VMEM((K, CW), jnp.float32),
        pltpu.VMEM((CW * K,), jnp.float32),
        pltpu.VMEM((CW * K,), jnp.float32),
        pltpu.VMEM((TAIL * K,), jnp.float32),
        pltpu.SemaphoreType.DMA,
        pltpu.SemaphoreType.DMA,
        pltpu.SemaphoreType.DMA,
        pltpu.SemaphoreType.DMA,
    ],
)
def _table_to_rowmajor(tt_hbm, tail_hbm, out_hbm, in_v0, in_v1,
                       out_v0, out_v1, tail_v,
                       in_sem0, in_sem1, out_sem0, out_sem1):
    wid = lax.axis_index("s") * NC + lax.axis_index("c")
    lanes32 = lax.iota(jnp.int32, L) * K
    in_bufs = (in_v0, in_v1)
    out_bufs = (out_v0, out_v1)
    in_sems = (in_sem0, in_sem1)
    out_sems = (out_sem0, out_sem1)

    def col0(i):
        return (wid * CPW + i) * CW

    def in_copy(s, i):
        return pltpu.make_async_copy(
            tt_hbm.at[:, pl.ds(col0(i), CW)], in_bufs[s], in_sems[s])

    def out_copy(s, i):
        return pltpu.make_async_copy(
            out_bufs[s], out_hbm.at[pl.ds(col0(i) * K, CW * K)], out_sems[s])

    def compute(s):
        def m_body(m, carry):
            idxm = lanes32 + m * (L * K)
            for k0 in range(0, K, 16):
                vecs = [in_bufs[s][k0 + d, pl.ds(m * L, L)] for d in range(16)]
                for d in range(16):
                    plsc.store_scatter(out_bufs[s], [idxm + (k0 + d)], vecs[d])
            return carry

        lax.fori_loop(0, CW // L, m_body, 0)

    in_copy(0, 0).start()
    in_copy(1, 1).start()

    def pair_body(j, carry):
        for s in range(2):
            i = 2 * j + s
            in_copy(s, i).wait()

            @pl.when(j > 0)
            def _drain():
                out_copy(s, i).wait()

            compute(s)
            out_copy(s, i).start()

            @pl.when(i + 2 <= CPW - 1)
            def _prefetch():
                in_copy(s, i + 2).start()
        return carry

    lax.fori_loop(0, (CPW - 1) // 2, pair_body, 0)

    last = CPW - 1
    in_copy(0, last).wait()
    out_copy(0, last).wait()
    compute(0)
    out_copy(0, last).start()

    @pl.when(wid == 0)
    def _extra():
        out_copy(1, 0).wait()
        ex = NW * CPW
        pltpu.sync_copy(tt_hbm.at[:, pl.ds(ex * CW, CW)], in_v1)
        compute(1)
        pltpu.async_copy(out_v1,
                         out_hbm.at[pl.ds(ex * CW * K, CW * K)],
                         out_sems[1]).wait()

    @pl.when(wid == 1)
    def _tail():
        pltpu.sync_copy(tail_hbm, tail_v)
        pltpu.sync_copy(tail_v, out_hbm.at[pl.ds(NFULL * CW * K, TAIL * K)])

    out_copy(0, last).wait()

    @pl.when(wid != 0)
    def _drain1():
        out_copy(1, 0).wait()


@functools.partial(
    pl.kernel,
    out_type=jax.ShapeDtypeStruct((B, K), jnp.float32),
    mesh=_mesh,
    compiler_params=pltpu.CompilerParams(use_tc_tiling_on_sc=False),
    scratch_types=[
        pltpu.VMEM((C * F,), jnp.int32),
        pltpu.VMEM((C * F,), jnp.int32),
        pltpu.VMEM((C, K), jnp.float32),
        pltpu.VMEM((C, K), jnp.float32),
        pltpu.VMEM((C * F, K), jnp.float32),
        pltpu.VMEM((C * F, K), jnp.float32),
        pltpu.VMEM((C, K), jnp.float32),
        pltpu.VMEM((C, K), jnp.float32),
        pltpu.SemaphoreType.DMA,
        pltpu.SemaphoreType.DMA,
        pltpu.SemaphoreType.DMA,
        pltpu.SemaphoreType.DMA,
    ],
)
def _fm_second_order(vals_hbm, idx_hbm, table_hbm, out_hbm,
                     idx_v0, idx_v1, vals_v0, vals_v1, rows_v0, rows_v1,
                     out_v0, out_v1, gsem0, gsem1, osem0, osem1):
    wid = lax.axis_index("s") * NC + lax.axis_index("c")
    base = wid * BPW
    idx_b = (idx_v0, idx_v1)
    vals_b = (vals_v0, vals_v1)
    rows_b = (rows_v0, rows_v1)
    out_b = (out_v0, out_v1)
    gsems = (gsem0, gsem1)
    osems = (osem0, osem1)

    def gather(s):
        return pltpu.make_async_copy(table_hbm.at[idx_b[s]], rows_b[s],
                                     gsems[s])

    def out_copy(s, g):
        return pltpu.make_async_copy(
            out_b[s], out_hbm.at[pl.ds(base + g * C, C)], osems[s])

    def load_and_gather(s, g):
        row0 = base + g * C
        pltpu.sync_copy(idx_hbm.at[pl.ds(row0 * F, C * F)], idx_b[s])
        pltpu.sync_copy(vals_hbm.at[pl.ds(row0, C)], vals_b[s])
        gather(s).start()

    def compute(s):
        def b_body(b, carry2):
            a1_lo = jnp.zeros((L,), jnp.float32)
            a1_hi = jnp.zeros((L,), jnp.float32)
            a2_lo = jnp.zeros((L,), jnp.float32)
            a2_hi = jnp.zeros((L,), jnp.float32)
            j0 = b * F
            v_lo = vals_b[s][b, pl.ds(0, L)]
            v_hi = vals_b[s][b, pl.ds(L, L)]
            for f in range(F):
                w = v_lo[f] if f < L else v_hi[f - L]
                e_lo = rows_b[s][j0 + f, pl.ds(0, L)]
                e_hi = rows_b[s][j0 + f, pl.ds(L, L)]
                we_lo = w * e_lo
                we_hi = w * e_hi
                a1_lo = a1_lo + we_lo
                a1_hi = a1_hi + we_hi
                a2_lo = a2_lo + we_lo * we_lo
                a2_hi = a2_hi + we_hi * we_hi
            out_b[s][b, pl.ds(0, L)] = a1_lo * a1_lo - a2_lo
            out_b[s][b, pl.ds(L, L)] = a1_hi * a1_hi - a2_hi
            return carry2

        lax.fori_loop(0, C, b_body, 0)

    load_and_gather(0, 0)

    def pair_body(j, carry):
        for s in range(2):
            i = 2 * j + s

            @pl.when(i + 1 <= NCHUNK - 1)
            def _prefetch():
                load_and_gather(1 - s, i + 1)

            gather(s).wait()

            @pl.when(i >= 2)
            def _drain():
                out_copy(s, i - 2).wait()

            compute(s)
            out_copy(s, i).start()
        return carry

    lax.fori_loop(0, NCHUNK // 2, pair_body, 0)
    out_copy(0, NCHUNK - 2).wait()
    out_copy(1, NCHUNK - 1).wait()


def kernel(feature_values, feature_idx, feature_embeddings):
    vals_padded = jnp.pad(feature_values, ((0, 0), (0, K - F)))
    table_flat = _table_to_rowmajor(
        feature_embeddings.T,
        feature_embeddings[NFULL * CW:].reshape(-1),
    )
    return _fm_second_order(
        vals_padded,
        feature_idx.reshape(-1),
        table_flat.reshape(V, K),
    )


# speedup vs baseline: 14.6698x; 1.7227x over previous
import functools

import jax
import jax.numpy as jnp
from jax import lax
from jax.experimental import pallas as pl
from jax.experimental.pallas import tpu as pltpu
from jax.experimental.pallas import tpu_sc as plsc

B = 16384
F = 26
K = 32
V = 1000000
NC = 2
NS = 16
NW = NC * NS
BPW = B // NW
C = 64
NCHUNK = BPW // C
L = 16

CW = 512
NFULL = V // CW
CPW = NFULL // NW
TAIL = V - NFULL * CW

_mesh = plsc.VectorSubcoreMesh(core_axis_name="c", subcore_axis_name="s")

KW = K // 2


def _pack_bf16_pair(lo, hi):
    lb = lax.bitcast_convert_type(lo, jnp.int32)
    hb = lax.bitcast_convert_type(hi, jnp.int32)
    lbr = lb + (jnp.int32(0x7FFF) + ((lb >> 16) & 1))
    hbr = hb + (jnp.int32(0x7FFF) + ((hb >> 16) & 1))
    return (hbr & jnp.int32(-65536)) | ((lbr >> 16) & jnp.int32(0xFFFF))


@functools.partial(
    pl.kernel,
    out_type=jax.ShapeDtypeStruct((V * KW,), jnp.int32),
    mesh=_mesh,
    compiler_params=pltpu.CompilerParams(
        use_tc_tiling_on_sc=True, needs_layout_passes=False),
    scratch_types=[
        pltpu.VMEM((K, CW), jnp.float32),
        pltpu.VMEM((K, CW), jnp.float32),
        pltpu.VMEM((CW * KW,), jnp.int32),
        pltpu.VMEM((CW * KW,), jnp.int32),
        pltpu.VMEM((TAIL * K,), jnp.float32),
        pltpu.VMEM((TAIL * KW,), jnp.int32),
        pltpu.SemaphoreType.DMA,
        pltpu.SemaphoreType.DMA,
        pltpu.SemaphoreType.DMA,
        pltpu.SemaphoreType.DMA,
    ],
)
def _table_to_rowmajor(tt_hbm, tail_hbm, out_hbm, in_v0, in_v1,
                       out_v0, out_v1, tail_v, tailp_v,
                       in_sem0, in_sem1, out_sem0, out_sem1):
    wid = lax.axis_index("s") * NC + lax.axis_index("c")
    lanes16 = lax.iota(jnp.int32, L) * KW
    in_bufs = (in_v0, in_v1)
    out_bufs = (out_v0, out_v1)
    in_sems = (in_sem0, in_sem1)
    out_sems = (out_sem0, out_sem1)

    def col0(i):
        return (wid * CPW + i) * CW

    def in_copy(s, i):
        return pltpu.make_async_copy(
            tt_hbm.at[:, pl.ds(col0(i), CW)], in_bufs[s], in_sems[s])

    def out_copy(s, i):
        return pltpu.make_async_copy(
            out_bufs[s], out_hbm.at[pl.ds(col0(i) * KW, CW * KW)], out_sems[s])

    def compute(s):
        def m_body(m, carry):
            idxm = lanes16 + m * (L * KW)
            for k in range(KW):
                lo = in_bufs[s][k, pl.ds(m * L, L)]
                hi = in_bufs[s][k + KW, pl.ds(m * L, L)]
                plsc.store_scatter(out_bufs[s], [idxm + k],
                                   _pack_bf16_pair(lo, hi))
            return carry

        lax.fori_loop(0, CW // L, m_body, 0)

    in_copy(0, 0).start()
    in_copy(1, 1).start()

    def pair_body(j, carry):
        for s in range(2):
            i = 2 * j + s
            in_copy(s, i).wait()

            @pl.when(j > 0)
            def _drain():
                out_copy(s, i).wait()

            compute(s)
            out_copy(s, i).start()

            @pl.when(i + 2 <= CPW - 1)
            def _prefetch():
                in_copy(s, i + 2).start()
        return carry

    lax.fori_loop(0, (CPW - 1) // 2, pair_body, 0)

    last = CPW - 1
    in_copy(0, last).wait()
    out_copy(0, last).wait()
    compute(0)
    out_copy(0, last).start()

    @pl.when(wid == 0)
    def _extra():
        out_copy(1, 0).wait()
        ex = NW * CPW
        pltpu.sync_copy(tt_hbm.at[:, pl.ds(ex * CW, CW)], in_v1)
        compute(1)
        pltpu.async_copy(out_v1,
                         out_hbm.at[pl.ds(ex * CW * KW, CW * KW)],
                         out_sems[1]).wait()

    @pl.when(wid == 1)
    def _tail():
        pltpu.sync_copy(tail_hbm, tail_v)

        def t_body(r, carry):
            lo = tail_v[pl.ds(r * K, L)]
            hi = tail_v[pl.ds(r * K + L, L)]
            tailp_v[pl.ds(r * KW, KW)] = _pack_bf16_pair(lo, hi)
            return carry

        lax.fori_loop(0, TAIL, t_body, 0)
        pltpu.sync_copy(tailp_v, out_hbm.at[pl.ds(NFULL * CW * KW, TAIL * KW)])

    out_copy(0, last).wait()

    @pl.when(wid != 0)
    def _drain1():
        out_copy(1, 0).wait()


@functools.partial(
    pl.kernel,
    out_type=jax.ShapeDtypeStruct((B, K), jnp.float32),
    mesh=_mesh,
    compiler_params=pltpu.CompilerParams(use_tc_tiling_on_sc=False),
    scratch_types=[
        pltpu.VMEM((C * F,), jnp.int32),
        pltpu.VMEM((C * F,), jnp.int32),
        pltpu.VMEM((C, K), jnp.float32),
        pltpu.VMEM((C, K), jnp.float32),
        pltpu.VMEM((C * F, KW), jnp.int32),
        pltpu.VMEM((C * F, KW), jnp.int32),
        pltpu.VMEM((C, K), jnp.float32),
        pltpu.VMEM((C, K), jnp.float32),
        pltpu.SemaphoreType.DMA,
        pltpu.SemaphoreType.DMA,
        pltpu.SemaphoreType.DMA,
        pltpu.SemaphoreType.DMA,
    ],
)
def _fm_second_order(vals_hbm, idx_hbm, table_hbm, out_hbm,
                     idx_v0, idx_v1, vals_v0, vals_v1, rows_v0, rows_v1,
                     out_v0, out_v1, gsem0, gsem1, osem0, osem1):
    wid = lax.axis_index("s") * NC + lax.axis_index("c")
    base = wid * BPW
    idx_b = (idx_v0, idx_v1)
    vals_b = (vals_v0, vals_v1)
    rows_b = (rows_v0, rows_v1)
    out_b = (out_v0, out_v1)
    gsems = (gsem0, gsem1)
    osems = (osem0, osem1)

    def gather(s):
        return pltpu.make_async_copy(table_hbm.at[idx_b[s]], rows_b[s],
                                     gsems[s])

    def out_copy(s, g):
        return pltpu.make_async_copy(
            out_b[s], out_hbm.at[pl.ds(base + g * C, C)], osems[s])

    def load_and_gather(s, g):
        row0 = base + g * C
        pltpu.sync_copy(idx_hbm.at[pl.ds(row0 * F, C * F)], idx_b[s])
        pltpu.sync_copy(vals_hbm.at[pl.ds(row0, C)], vals_b[s])
        gather(s).start()

    def compute(s):
        def b_body(b, carry2):
            a1_lo = jnp.zeros((L,), jnp.float32)
            a1_hi = jnp.zeros((L,), jnp.float32)
            a2_lo = jnp.zeros((L,), jnp.float32)
            a2_hi = jnp.zeros((L,), jnp.float32)
            j0 = b * F
            v_lo = vals_b[s][b, pl.ds(0, L)]
            v_hi = vals_b[s][b, pl.ds(L, L)]
            for f in range(F):
                w = v_lo[f] if f < L else v_hi[f - L]
                pk = rows_b[s][j0 + f, pl.ds(0, KW)]
                e_lo = lax.bitcast_convert_type(pk << 16, jnp.float32)
                e_hi = lax.bitcast_convert_type(
                    pk & jnp.int32(-65536), jnp.float32)
                we_lo = w * e_lo
                we_hi = w * e_hi
                a1_lo = a1_lo + we_lo
                a1_hi = a1_hi + we_hi
                a2_lo = a2_lo + we_lo * we_lo
                a2_hi = a2_hi + we_hi * we_hi
            out_b[s][b, pl.ds(0, L)] = a1_lo * a1_lo - a2_lo
            out_b[s][b, pl.ds(L, L)] = a1_hi * a1_hi - a2_hi
            return carry2

        lax.fori_loop(0, C, b_body, 0)

    load_and_gather(0, 0)

    def pair_body(j, carry):
        for s in range(2):
            i = 2 * j + s

            @pl.when(i + 1 <= NCHUNK - 1)
            def _prefetch():
                load_and_gather(1 - s, i + 1)

            gather(s).wait()

            @pl.when(i >= 2)
            def _drain():
                out_copy(s, i - 2).wait()

            compute(s)
            out_copy(s, i).start()
        return carry

    lax.fori_loop(0, NCHUNK // 2, pair_body, 0)
    out_copy(0, NCHUNK - 2).wait()
    out_copy(1, NCHUNK - 1).wait()


def kernel(feature_values, feature_idx, feature_embeddings):
    vals_padded = jnp.pad(feature_values, ((0, 0), (0, K - F)))
    table_flat = _table_to_rowmajor(
        feature_embeddings.T,
        feature_embeddings[NFULL * CW:].reshape(-1),
    )
    return _fm_second_order(
        vals_padded,
        feature_idx.reshape(-1),
        table_flat.reshape(V, KW),
    )
